# baseline (speedup 1.0000x reference)
import functools

import jax
import jax.numpy as jnp
from jax import lax
from jax.experimental import pallas as pl
from jax.experimental.pallas import tpu as pltpu
from jax.experimental.pallas import tpu_sc as plsc

K_NN = 16
W = 128
TQ_A = 512
TN = 2048
TQ_B = 256
TQ_D = 256
BIG = 1e30
BIG2 = 2e30
BIGI = 1e9

NC_SC = 2
NS_SC = 16
NW_SC = NC_SC * NS_SC


def _dist_body(n_real, q_ref, k_ref, d2_ref, gm_ref):
    q = q_ref[...]
    kk = k_ref[...]
    q2 = jnp.sum(q * q, axis=1, keepdims=True)
    k2 = jnp.sum(kk * kk, axis=1, keepdims=True)
    dot = lax.dot_general(q, kk, (((1,), (1,)), ((), ())),
                          preferred_element_type=jnp.float32)
    d2 = q2 + k2[None, :, 0] - 2.0 * dot
    ni = pl.program_id(1)
    gcols = []
    for t in range(TN // 256):
        dt = d2[:, t * 256:(t + 1) * 256]
        col = ni * TN + t * 256 + lax.broadcasted_iota(
            jnp.int32, (TQ_A, 256), 1)
        dt = jnp.where(col < n_real, dt, BIG)
        d2_ref[0, 2 * t] = dt[:, :W]
        d2_ref[0, 2 * t + 1] = dt[:, W:]
        gcols.append(jnp.min(dt[:, :W], axis=1, keepdims=True))
        gcols.append(jnp.min(dt[:, W:], axis=1, keepdims=True))
    gm_ref[0, :, :] = jnp.concatenate(gcols, axis=1)


def _group_topk_body(n_groups, q_n, gm_ref, gids_ref, fidx_ref):
    vals = gm_ref[...]
    gio = lax.broadcasted_iota(jnp.int32, vals.shape, 1).astype(jnp.float32)
    cols = []
    for _ in range(K_NN):
        m = jnp.min(vals, axis=1, keepdims=True)
        sel = jnp.min(jnp.where(vals <= m, gio, BIGI), axis=1, keepdims=True)
        cols.append(sel)
        vals = jnp.where(gio == sel, BIG2, vals)
    gids = jnp.concatenate(cols, axis=1).astype(jnp.int32)
    row = pl.program_id(0) * TQ_B + lax.broadcasted_iota(
        jnp.int32, (TQ_B, K_NN), 0)
    gids_ref[...] = gids
    fidx_ref[...] = gids * q_n + row


def _final_body(cand_ref, gids_ref, dist_ref, idx_ref):
    vals = cand_ref[...]
    gids = gids_ref[...]
    lane = lax.broadcasted_iota(jnp.int32, (TQ_D, W), 1)
    gidx = jnp.concatenate(
        [(gids[:, s:s + 1] * W + lane).astype(jnp.float32)
         for s in range(K_NN)], axis=1)
    dcols, icols = [], []
    for _ in range(K_NN):
        m = jnp.min(vals, axis=1, keepdims=True)
        cidx = jnp.min(jnp.where(vals <= m, gidx, BIGI), axis=1, keepdims=True)
        dcols.append(m)
        icols.append(cidx)
        vals = jnp.where(gidx == cidx, BIG2, vals)
    dist_ref[...] = jnp.concatenate(dcols, axis=1)
    idx_ref[...] = jnp.concatenate(icols, axis=1).astype(jnp.int32)


def _sc_gather(table, idx_flat):
    b_rows = idx_flat.shape[0]
    b_per_w = b_rows // NW_SC
    chunk = 128
    n_ch = b_per_w // chunk
    mesh = plsc.VectorSubcoreMesh(core_axis_name="c", subcore_axis_name="s")

    @functools.partial(
        pl.kernel,
        out_type=jax.ShapeDtypeStruct((b_rows, W), jnp.float32),
        mesh=mesh,
        scratch_types=[
            pltpu.VMEM((chunk,), jnp.int32),
            pltpu.VMEM((chunk, W), jnp.float32),
            pltpu.SemaphoreType.DMA,
        ],
    )
    def gather_kernel(table_hbm, idx_hbm, out_hbm, idx_v, rows_v, sem):
        wid = lax.axis_index("s") * NC_SC + lax.axis_index("c")
        base = wid * b_per_w

        def body(i, carry):
            off = base + i * chunk
            pltpu.sync_copy(idx_hbm.at[pl.ds(off, chunk)], idx_v)
            pltpu.async_copy(table_hbm.at[idx_v], rows_v, sem).wait()
            pltpu.sync_copy(rows_v, out_hbm.at[pl.ds(off, chunk)])
            return carry

        lax.fori_loop(0, n_ch, body, 0)

    return gather_kernel(table, idx_flat)


def kernel(queries, keys, k):
    q_n, d = queries.shape
    n_real = keys.shape[0]
    n_pad = ((n_real + TN - 1) // TN) * TN
    n_groups = n_pad // W
    n_tiles_n = n_pad // TN
    n_tiles_qa = q_n // TQ_A

    keys_p = jnp.pad(keys, ((0, n_pad - n_real), (0, 0)))

    d2t, gm3 = pl.pallas_call(
        functools.partial(_dist_body, n_real),
        grid=(n_tiles_qa, n_tiles_n),
        in_specs=[
            pl.BlockSpec((TQ_A, d), lambda qi, ni: (qi, 0)),
            pl.BlockSpec((TN, d), lambda qi, ni: (ni, 0)),
        ],
        out_specs=[
            pl.BlockSpec((1, TN // W, TQ_A, W), lambda qi, ni: (ni, 0, qi, 0)),
            pl.BlockSpec((1, TQ_A, TN // W), lambda qi, ni: (ni, qi, 0)),
        ],
        out_shape=[
            jax.ShapeDtypeStruct((n_tiles_n, TN // W, q_n, W), jnp.float32),
            jax.ShapeDtypeStruct((n_tiles_n, q_n, TN // W), jnp.float32),
        ],
        compiler_params=pltpu.CompilerParams(
            dimension_semantics=("parallel", "arbitrary")),
    )(queries, keys_p)

    gmins = gm3.transpose(1, 0, 2).reshape(q_n, n_groups)

    gids, fidx = pl.pallas_call(
        functools.partial(_group_topk_body, n_groups, q_n),
        grid=(q_n // TQ_B,),
        in_specs=[pl.BlockSpec((TQ_B, n_groups), lambda i: (i, 0))],
        out_specs=[
            pl.BlockSpec((TQ_B, K_NN), lambda i: (i, 0)),
            pl.BlockSpec((TQ_B, K_NN), lambda i: (i, 0)),
        ],
        out_shape=[
            jax.ShapeDtypeStruct((q_n, K_NN), jnp.int32),
            jax.ShapeDtypeStruct((q_n, K_NN), jnp.int32),
        ],
    )(gmins)

    table = d2t.reshape(q_n * n_groups, W)
    cand = _sc_gather(table, fidx.reshape(q_n * K_NN))
    cand2 = cand.reshape(q_n, K_NN * W)

    dists, idx = pl.pallas_call(
        _final_body,
        grid=(q_n // TQ_D,),
        in_specs=[
            pl.BlockSpec((TQ_D, K_NN * W), lambda i: (i, 0)),
            pl.BlockSpec((TQ_D, K_NN), lambda i: (i, 0)),
        ],
        out_specs=[
            pl.BlockSpec((TQ_D, K_NN), lambda i: (i, 0)),
            pl.BlockSpec((TQ_D, K_NN), lambda i: (i, 0)),
        ],
        out_shape=[
            jax.ShapeDtypeStruct((q_n, K_NN), jnp.float32),
            jax.ShapeDtypeStruct((q_n, K_NN), jnp.int32),
        ],
    )(cand2, gids)

    return dists, idx


# speedup vs baseline: 1.1804x; 1.1804x over previous
import functools

import jax
import jax.numpy as jnp
from jax import lax
from jax.experimental import pallas as pl
from jax.experimental.pallas import tpu as pltpu
from jax.experimental.pallas import tpu_sc as plsc

K_NN = 16
W = 128
TQ_A = 2048
TN = 2048
TQ_B = 1024
TQ_D = 1024
BIG = 1e30
BIG2 = 2e30
BIGI = 1e9

NC_SC = 2
NS_SC = 16
NW_SC = NC_SC * NS_SC


def _dist_body(n_real, q_ref, k_ref, d2_ref, gm_ref):
    q = q_ref[...]
    kk = k_ref[...]
    q2 = jnp.sum(q * q, axis=1, keepdims=True)
    k2 = jnp.sum(kk * kk, axis=1, keepdims=True)
    dot = lax.dot_general(q, kk, (((1,), (1,)), ((), ())),
                          preferred_element_type=jnp.float32)
    d2 = q2 + k2[None, :, 0] - 2.0 * dot
    ni = pl.program_id(1)
    gcols = []
    for t in range(TN // 256):
        dt = d2[:, t * 256:(t + 1) * 256]
        col = ni * TN + t * 256 + lax.broadcasted_iota(
            jnp.int32, (TQ_A, 256), 1)
        dt = jnp.where(col < n_real, dt, BIG)
        d2_ref[0, 2 * t] = dt[:, :W]
        d2_ref[0, 2 * t + 1] = dt[:, W:]
        gcols.append(jnp.min(dt[:, :W], axis=1, keepdims=True))
        gcols.append(jnp.min(dt[:, W:], axis=1, keepdims=True))
    gm_ref[0, :, :] = jnp.concatenate(gcols, axis=1)


def _group_topk_body(n_groups, q_n, gm_ref, gids_ref, fidx_ref):
    vals = gm_ref[...]
    gio = lax.broadcasted_iota(jnp.int32, vals.shape, 1).astype(jnp.float32)
    cols = []
    for _ in range(K_NN):
        m = jnp.min(vals, axis=1, keepdims=True)
        sel = jnp.min(jnp.where(vals <= m, gio, BIGI), axis=1, keepdims=True)
        cols.append(sel)
        vals = jnp.where(gio == sel, BIG2, vals)
    gids = jnp.concatenate(cols, axis=1).astype(jnp.int32)
    row = pl.program_id(0) * TQ_B + lax.broadcasted_iota(
        jnp.int32, (TQ_B, K_NN), 0)
    gids_ref[...] = gids
    fidx_ref[...] = gids * q_n + row


def _final_body(cand_ref, gids_ref, dist_ref, idx_ref):
    vals = cand_ref[...]
    gids = gids_ref[...]
    lane = lax.broadcasted_iota(jnp.int32, (TQ_D, W), 1)
    gidx = jnp.concatenate(
        [(gids[:, s:s + 1] * W + lane).astype(jnp.float32)
         for s in range(K_NN)], axis=1)
    dcols, icols = [], []
    for _ in range(K_NN):
        m = jnp.min(vals, axis=1, keepdims=True)
        cidx = jnp.min(jnp.where(vals <= m, gidx, BIGI), axis=1, keepdims=True)
        dcols.append(m)
        icols.append(cidx)
        vals = jnp.where(gidx == cidx, BIG2, vals)
    dist_ref[...] = jnp.concatenate(dcols, axis=1)
    idx_ref[...] = jnp.concatenate(icols, axis=1).astype(jnp.int32)


def _sc_gather(table, idx_flat):
    b_rows = idx_flat.shape[0]
    b_per_w = b_rows // NW_SC
    chunk = 128
    n_ch = b_per_w // chunk
    mesh = plsc.VectorSubcoreMesh(core_axis_name="c", subcore_axis_name="s")

    @functools.partial(
        pl.kernel,
        out_type=jax.ShapeDtypeStruct((b_rows, W), jnp.float32),
        mesh=mesh,
        scratch_types=[
            pltpu.VMEM((chunk,), jnp.int32),
            pltpu.VMEM((chunk, W), jnp.float32),
            pltpu.SemaphoreType.DMA,
        ],
    )
    def gather_kernel(table_hbm, idx_hbm, out_hbm, idx_v, rows_v, sem):
        wid = lax.axis_index("s") * NC_SC + lax.axis_index("c")
        base = wid * b_per_w

        def body(i, carry):
            off = base + i * chunk
            pltpu.sync_copy(idx_hbm.at[pl.ds(off, chunk)], idx_v)
            pltpu.async_copy(table_hbm.at[idx_v], rows_v, sem).wait()
            pltpu.sync_copy(rows_v, out_hbm.at[pl.ds(off, chunk)])
            return carry

        lax.fori_loop(0, n_ch, body, 0)

    return gather_kernel(table, idx_flat)


def kernel(queries, keys, k):
    q_n, d = queries.shape
    n_real = keys.shape[0]
    n_pad = ((n_real + TN - 1) // TN) * TN
    n_groups = n_pad // W
    n_tiles_n = n_pad // TN
    n_tiles_qa = q_n // TQ_A

    keys_p = jnp.pad(keys, ((0, n_pad - n_real), (0, 0)))

    d2t, gm3 = pl.pallas_call(
        functools.partial(_dist_body, n_real),
        grid=(n_tiles_qa, n_tiles_n),
        in_specs=[
            pl.BlockSpec((TQ_A, d), lambda qi, ni: (qi, 0)),
            pl.BlockSpec((TN, d), lambda qi, ni: (ni, 0)),
        ],
        out_specs=[
            pl.BlockSpec((1, TN // W, TQ_A, W), lambda qi, ni: (ni, 0, qi, 0)),
            pl.BlockSpec((1, TQ_A, TN // W), lambda qi, ni: (ni, qi, 0)),
        ],
        out_shape=[
            jax.ShapeDtypeStruct((n_tiles_n, TN // W, q_n, W), jnp.float32),
            jax.ShapeDtypeStruct((n_tiles_n, q_n, TN // W), jnp.float32),
        ],
        compiler_params=pltpu.CompilerParams(
            dimension_semantics=("parallel", "arbitrary")),
    )(queries, keys_p)

    gmins = gm3.transpose(1, 0, 2).reshape(q_n, n_groups)

    gids, fidx = pl.pallas_call(
        functools.partial(_group_topk_body, n_groups, q_n),
        grid=(q_n // TQ_B,),
        in_specs=[pl.BlockSpec((TQ_B, n_groups), lambda i: (i, 0))],
        out_specs=[
            pl.BlockSpec((TQ_B, K_NN), lambda i: (i, 0)),
            pl.BlockSpec((TQ_B, K_NN), lambda i: (i, 0)),
        ],
        out_shape=[
            jax.ShapeDtypeStruct((q_n, K_NN), jnp.int32),
            jax.ShapeDtypeStruct((q_n, K_NN), jnp.int32),
        ],
    )(gmins)

    table = d2t.reshape(q_n * n_groups, W)
    cand = _sc_gather(table, fidx.reshape(q_n * K_NN))
    cand2 = cand.reshape(q_n, K_NN * W)

    dists, idx = pl.pallas_call(
        _final_body,
        grid=(q_n // TQ_D,),
        in_specs=[
            pl.BlockSpec((TQ_D, K_NN * W), lambda i: (i, 0)),
            pl.BlockSpec((TQ_D, K_NN), lambda i: (i, 0)),
        ],
        out_specs=[
            pl.BlockSpec((TQ_D, K_NN), lambda i: (i, 0)),
            pl.BlockSpec((TQ_D, K_NN), lambda i: (i, 0)),
        ],
        out_shape=[
            jax.ShapeDtypeStruct((q_n, K_NN), jnp.float32),
            jax.ShapeDtypeStruct((q_n, K_NN), jnp.int32),
        ],
    )(cand2, gids)

    return dists, idx


# speedup vs baseline: 1.1970x; 1.0141x over previous
import functools

import jax
import jax.numpy as jnp
from jax import lax
from jax.experimental import pallas as pl
from jax.experimental.pallas import tpu as pltpu
from jax.experimental.pallas import tpu_sc as plsc

K_NN = 16
W = 128
TQ_A = 1024
TN = 2048
TQ_B = 512
TQ_D = 512
BIG = 1e30
BIG2 = 2e30
BIGI = 1e9

NC_SC = 2
NS_SC = 16
NW_SC = NC_SC * NS_SC


def _dist_body(n_real, q_ref, k_ref, d2_ref, gm_ref):
    q = q_ref[...]
    kk = k_ref[...]
    q2 = jnp.sum(q * q, axis=1, keepdims=True)
    k2 = jnp.sum(kk * kk, axis=1, keepdims=True)
    dot = lax.dot_general(q, kk, (((1,), (1,)), ((), ())),
                          preferred_element_type=jnp.float32)
    d2 = q2 + k2[None, :, 0] - 2.0 * dot
    ni = pl.program_id(1)
    gcols = []
    for t in range(TN // 256):
        dt = d2[:, t * 256:(t + 1) * 256]
        col = ni * TN + t * 256 + lax.broadcasted_iota(
            jnp.int32, (TQ_A, 256), 1)
        dt = jnp.where(col < n_real, dt, BIG)
        d2_ref[0, 2 * t] = dt[:, :W]
        d2_ref[0, 2 * t + 1] = dt[:, W:]
        gcols.append(jnp.min(dt[:, :W], axis=1, keepdims=True))
        gcols.append(jnp.min(dt[:, W:], axis=1, keepdims=True))
    gm_ref[0, :, :] = jnp.concatenate(gcols, axis=1)


def _group_topk_body(n_groups, q_n, gm_ref, gids_ref, fidx_ref):
    vals = gm_ref[...]
    gio = lax.broadcasted_iota(jnp.int32, vals.shape, 1).astype(jnp.float32)
    cols = []
    for _ in range(K_NN):
        m = jnp.min(vals, axis=1, keepdims=True)
        sel = jnp.min(jnp.where(vals <= m, gio, BIGI), axis=1, keepdims=True)
        cols.append(sel)
        vals = jnp.where(gio == sel, BIG2, vals)
    gids = jnp.concatenate(cols, axis=1).astype(jnp.int32)
    row = pl.program_id(0) * TQ_B + lax.broadcasted_iota(
        jnp.int32, (TQ_B, K_NN), 0)
    gids_ref[...] = gids
    fidx_ref[...] = gids * q_n + row


def _final_body(cand_ref, gids_ref, dist_ref, idx_ref):
    vals = cand_ref[...]
    gids = gids_ref[...]
    lane = lax.broadcasted_iota(jnp.int32, (TQ_D, W), 1)
    gidx = jnp.concatenate(
        [(gids[:, s:s + 1] * W + lane).astype(jnp.float32)
         for s in range(K_NN)], axis=1)
    dcols, icols = [], []
    for _ in range(K_NN):
        m = jnp.min(vals, axis=1, keepdims=True)
        cidx = jnp.min(jnp.where(vals <= m, gidx, BIGI), axis=1, keepdims=True)
        dcols.append(m)
        icols.append(cidx)
        vals = jnp.where(gidx == cidx, BIG2, vals)
    dist_ref[...] = jnp.concatenate(dcols, axis=1)
    idx_ref[...] = jnp.concatenate(icols, axis=1).astype(jnp.int32)


def _sc_gather(table, idx_flat):
    b_rows = idx_flat.shape[0]
    b_per_w = b_rows // NW_SC
    chunk = 128
    n_ch = b_per_w // chunk
    mesh = plsc.VectorSubcoreMesh(core_axis_name="c", subcore_axis_name="s")

    @functools.partial(
        pl.kernel,
        out_type=jax.ShapeDtypeStruct((b_rows, W), jnp.float32),
        mesh=mesh,
        scratch_types=[
            pltpu.VMEM((chunk,), jnp.int32),
            pltpu.VMEM((chunk, W), jnp.float32),
            pltpu.SemaphoreType.DMA,
        ],
    )
    def gather_kernel(table_hbm, idx_hbm, out_hbm, idx_v, rows_v, sem):
        wid = lax.axis_index("s") * NC_SC + lax.axis_index("c")
        base = wid * b_per_w

        def body(i, carry):
            off = base + i * chunk
            pltpu.sync_copy(idx_hbm.at[pl.ds(off, chunk)], idx_v)
            pltpu.async_copy(table_hbm.at[idx_v], rows_v, sem).wait()
            pltpu.sync_copy(rows_v, out_hbm.at[pl.ds(off, chunk)])
            return carry

        lax.fori_loop(0, n_ch, body, 0)

    return gather_kernel(table, idx_flat)


def kernel(queries, keys, k):
    q_n, d = queries.shape
    n_real = keys.shape[0]
    n_pad = ((n_real + TN - 1) // TN) * TN
    n_groups = n_pad // W
    n_tiles_n = n_pad // TN
    n_tiles_qa = q_n // TQ_A

    keys_p = jnp.pad(keys, ((0, n_pad - n_real), (0, 0)))

    d2t, gm3 = pl.pallas_call(
        functools.partial(_dist_body, n_real),
        grid=(n_tiles_qa, n_tiles_n),
        in_specs=[
            pl.BlockSpec((TQ_A, d), lambda qi, ni: (qi, 0)),
            pl.BlockSpec((TN, d), lambda qi, ni: (ni, 0)),
        ],
        out_specs=[
            pl.BlockSpec((1, TN // W, TQ_A, W), lambda qi, ni: (ni, 0, qi, 0)),
            pl.BlockSpec((1, TQ_A, TN // W), lambda qi, ni: (ni, qi, 0)),
        ],
        out_shape=[
            jax.ShapeDtypeStruct((n_tiles_n, TN // W, q_n, W), jnp.float32),
            jax.ShapeDtypeStruct((n_tiles_n, q_n, TN // W), jnp.float32),
        ],
        compiler_params=pltpu.CompilerParams(
            dimension_semantics=("parallel", "arbitrary")),
    )(queries, keys_p)

    gmins = gm3.transpose(1, 0, 2).reshape(q_n, n_groups)

    gids, fidx = pl.pallas_call(
        functools.partial(_group_topk_body, n_groups, q_n),
        grid=(q_n // TQ_B,),
        in_specs=[pl.BlockSpec((TQ_B, n_groups), lambda i: (i, 0))],
        out_specs=[
            pl.BlockSpec((TQ_B, K_NN), lambda i: (i, 0)),
            pl.BlockSpec((TQ_B, K_NN), lambda i: (i, 0)),
        ],
        out_shape=[
            jax.ShapeDtypeStruct((q_n, K_NN), jnp.int32),
            jax.ShapeDtypeStruct((q_n, K_NN), jnp.int32),
        ],
    )(gmins)

    table = d2t.reshape(q_n * n_groups, W)
    cand = _sc_gather(table, fidx.reshape(q_n * K_NN))
    cand2 = cand.reshape(q_n, K_NN * W)

    dists, idx = pl.pallas_call(
        _final_body,
        grid=(q_n // TQ_D,),
        in_specs=[
            pl.BlockSpec((TQ_D, K_NN * W), lambda i: (i, 0)),
            pl.BlockSpec((TQ_D, K_NN), lambda i: (i, 0)),
        ],
        out_specs=[
            pl.BlockSpec((TQ_D, K_NN), lambda i: (i, 0)),
            pl.BlockSpec((TQ_D, K_NN), lambda i: (i, 0)),
        ],
        out_shape=[
            jax.ShapeDtypeStruct((q_n, K_NN), jnp.float32),
            jax.ShapeDtypeStruct((q_n, K_NN), jnp.int32),
        ],
    )(cand2, gids)

    return dists, idx
